# baseline (device time: 208412 ns/iter reference)
import jax
import jax.numpy as jnp
from jax import lax
from jax.experimental import pallas as pl
from jax.experimental.pallas import tpu as pltpu

N_DEV = 4
TOK = 512
D = 512
F = 1024
E_LOC = 2
CAP = 160


def kernel(x, assign, W1, W2):
    dest = assign // 2
    idx = jnp.stack(
        [jnp.nonzero(dest == d, size=CAP, fill_value=TOK)[0] for d in range(N_DEV)]
    )
    xp = jnp.concatenate([x, jnp.zeros((1, D), x.dtype)], axis=0)
    ap = jnp.concatenate([assign, jnp.full((1,), -1, assign.dtype)], axis=0)
    xs = xp[idx].astype(jnp.bfloat16)
    tags = ap[idx][..., None]

    def body(xs_ref, tg_ref, w1_ref, w2_ref, out_ref,
             xin, tin, rout, w1b, w2b,
             d_s, d_r, t_s, t_r, r_s, r_r):
        my = lax.axis_index("i")
        left = (my + N_DEV - 1) % N_DEV
        right = (my + 1) % N_DEV
        opp = (my + 2) % N_DEV

        barrier_sem = pltpu.get_barrier_semaphore()
        for nbr in (left, right, opp):
            pl.semaphore_signal(
                barrier_sem, inc=1,
                device_id=(nbr,), device_id_type=pl.DeviceIdType.MESH,
            )
        pl.semaphore_wait(barrier_sem, 3)

        def copy(src_ref, dst_ref, send_sem, recv_sem, dst_dev):
            rdma = pltpu.make_async_remote_copy(
                src_ref=src_ref, dst_ref=dst_ref,
                send_sem=send_sem, recv_sem=recv_sem,
                device_id=(dst_dev,), device_id_type=pl.DeviceIdType.MESH,
            )
            rdma.start()
            return rdma

        d0 = copy(xs_ref.at[right], xin.at[my], d_s.at[0], d_r.at[0], right)
        d1 = copy(xs_ref.at[left], xin.at[my], d_s.at[1], d_r.at[1], left)
        d2 = copy(xs_ref.at[opp], xin.at[my], d_s.at[2], d_r.at[2], opp)
        t0 = copy(tg_ref.at[right], tin.at[my], t_s.at[0], t_r.at[0], right)
        t1 = copy(tg_ref.at[left], tin.at[my], t_s.at[1], t_r.at[1], left)
        t2 = copy(tg_ref.at[opp], tin.at[my], t_s.at[2], t_r.at[2], opp)

        w1b[...] = w1_ref[...].astype(jnp.bfloat16)
        w2b[...] = w2_ref[...].astype(jnp.bfloat16)

        def compute_chunk(s):
            xb = xin[s]
            tb = tin[s]
            acc = jnp.zeros((CAP, D), dtype=jnp.float32)
            for e in range(E_LOC):
                h_act = jnp.maximum(
                    jnp.dot(xb, w1b[e], preferred_element_type=jnp.float32),
                    0.0,
                ).astype(jnp.bfloat16)
                y = jnp.dot(h_act, w2b[e], preferred_element_type=jnp.float32)
                acc = acc + jnp.where(tb == my * E_LOC + e, y, 0.0)
            rout[s] = acc.astype(jnp.bfloat16)

        xin[my] = xs_ref[my]
        tin[my] = tg_ref[my]
        compute_chunk(my)
        out_ref[my] = rout[my]

        d0.wait()
        t0.wait()
        compute_chunk(left)
        r0 = copy(rout.at[left], out_ref.at[my], r_s.at[0], r_r.at[0], left)

        d1.wait()
        t1.wait()
        compute_chunk(right)
        r1 = copy(rout.at[right], out_ref.at[my], r_s.at[1], r_r.at[1], right)

        d2.wait()
        t2.wait()
        compute_chunk(opp)
        r2 = copy(rout.at[opp], out_ref.at[my], r_s.at[2], r_r.at[2], opp)

        r0.wait()
        r1.wait()
        r2.wait()

    rb = pl.pallas_call(
        body,
        out_shape=jax.ShapeDtypeStruct((N_DEV, CAP, D), jnp.bfloat16),
        in_specs=[
            pl.BlockSpec(memory_space=pltpu.VMEM),
            pl.BlockSpec(memory_space=pltpu.VMEM),
            pl.BlockSpec(memory_space=pltpu.VMEM),
            pl.BlockSpec(memory_space=pltpu.VMEM),
        ],
        out_specs=pl.BlockSpec(memory_space=pltpu.VMEM),
        scratch_shapes=[
            pltpu.VMEM((N_DEV, CAP, D), jnp.bfloat16),
            pltpu.VMEM((N_DEV, CAP, 1), jnp.int32),
            pltpu.VMEM((N_DEV, CAP, D), jnp.bfloat16),
            pltpu.VMEM((E_LOC, D, F), jnp.bfloat16),
            pltpu.VMEM((E_LOC, F, D), jnp.bfloat16),
            pltpu.SemaphoreType.DMA((3,)),
            pltpu.SemaphoreType.DMA((3,)),
            pltpu.SemaphoreType.DMA((3,)),
            pltpu.SemaphoreType.DMA((3,)),
            pltpu.SemaphoreType.DMA((3,)),
            pltpu.SemaphoreType.DMA((3,)),
        ],
        compiler_params=pltpu.CompilerParams(collective_id=0),
    )(xs, tags, W1, W2)

    outp = jnp.zeros((TOK + 1, D), jnp.float32)
    outp = outp.at[idx.reshape(-1)].set(rb.astype(jnp.float32).reshape(-1, D))
    return outp[:TOK]


# device time: 27743 ns/iter; 7.5122x vs baseline; 7.5122x over previous
import jax
import jax.numpy as jnp
from jax import lax
from jax.experimental import pallas as pl
from jax.experimental.pallas import tpu as pltpu

N_DEV = 4
TOK = 512
D = 512
F = 1024
E_LOC = 2
CAP = 160
SLOTS = N_DEV * CAP


def kernel(x, assign, W1, W2):
    dest = assign // 2
    rp = jnp.zeros((TOK,), jnp.int32)
    for d in range(N_DEV):
        m = dest == d
        pos = jnp.cumsum(m.astype(jnp.int32)) - 1
        rp = jnp.where(m, d * CAP + pos, rp)
    rp_col = rp[:, None]
    rp_row = rp[None, :]
    assign2d = assign[:, None]

    def body(x_ref, a_ref, rpc_ref, rpr_ref, w1_ref, w2_ref, out_ref,
             xs, ts, xin, tin, rout, rin, w1b, w2b,
             d_s, d_r, t_s, t_r, r_s, r_r):
        my = lax.axis_index("i")
        left = (my + N_DEV - 1) % N_DEV
        right = (my + 1) % N_DEV
        opp = (my + 2) % N_DEV

        barrier_sem = pltpu.get_barrier_semaphore()
        for nbr in (left, right, opp):
            pl.semaphore_signal(
                barrier_sem, inc=1,
                device_id=(nbr,), device_id_type=pl.DeviceIdType.MESH,
            )
        pl.semaphore_wait(barrier_sem, 3)

        ohT = (
            rpr_ref[...] == lax.broadcasted_iota(jnp.int32, (SLOTS, TOK), 0)
        ).astype(jnp.bfloat16)
        xb16 = x_ref[...].astype(jnp.bfloat16)
        xs[...] = jnp.dot(
            ohT, xb16, preferred_element_type=jnp.float32
        ).astype(jnp.bfloat16)
        tagv = (a_ref[...] + 1).astype(jnp.bfloat16)
        ts[...] = jnp.dot(
            ohT, tagv, preferred_element_type=jnp.float32
        ).astype(jnp.bfloat16)

        def copy(src_ref, dst_ref, send_sem, recv_sem, dst_dev):
            rdma = pltpu.make_async_remote_copy(
                src_ref=src_ref, dst_ref=dst_ref,
                send_sem=send_sem, recv_sem=recv_sem,
                device_id=(dst_dev,), device_id_type=pl.DeviceIdType.MESH,
            )
            rdma.start()
            return rdma

        def chunk(ref, s):
            return ref.at[pl.ds(s * CAP, CAP), :]

        d0 = copy(chunk(xs, right), chunk(xin, my), d_s.at[0], d_r.at[0], right)
        d1 = copy(chunk(xs, left), chunk(xin, my), d_s.at[1], d_r.at[1], left)
        d2 = copy(chunk(xs, opp), chunk(xin, my), d_s.at[2], d_r.at[2], opp)
        t0 = copy(chunk(ts, right), chunk(tin, my), t_s.at[0], t_r.at[0], right)
        t1 = copy(chunk(ts, left), chunk(tin, my), t_s.at[1], t_r.at[1], left)
        t2 = copy(chunk(ts, opp), chunk(tin, my), t_s.at[2], t_r.at[2], opp)

        w1b[...] = w1_ref[...].astype(jnp.bfloat16)
        w2b[...] = w2_ref[...].astype(jnp.bfloat16)

        def compute_chunk(s):
            xb = xin[pl.ds(s * CAP, CAP), :]
            tb = tin[pl.ds(s * CAP, CAP), :]
            acc = jnp.zeros((CAP, D), dtype=jnp.float32)
            for e in range(E_LOC):
                h_act = jnp.maximum(
                    jnp.dot(xb, w1b[e], preferred_element_type=jnp.float32),
                    0.0,
                ).astype(jnp.bfloat16)
                y = jnp.dot(h_act, w2b[e], preferred_element_type=jnp.float32)
                tag_e = (my * E_LOC + e + 1).astype(jnp.bfloat16)
                acc = acc + jnp.where(tb == tag_e, y, 0.0)
            rout[pl.ds(s * CAP, CAP), :] = acc.astype(jnp.bfloat16)

        xin[pl.ds(my * CAP, CAP), :] = xs[pl.ds(my * CAP, CAP), :]
        tin[pl.ds(my * CAP, CAP), :] = ts[pl.ds(my * CAP, CAP), :]
        compute_chunk(my)
        rin[pl.ds(my * CAP, CAP), :] = rout[pl.ds(my * CAP, CAP), :]

        d0.wait()
        t0.wait()
        compute_chunk(left)
        r0 = copy(chunk(rout, left), chunk(rin, my), r_s.at[0], r_r.at[0], left)

        d1.wait()
        t1.wait()
        compute_chunk(right)
        r1 = copy(chunk(rout, right), chunk(rin, my), r_s.at[1], r_r.at[1], right)

        d2.wait()
        t2.wait()
        compute_chunk(opp)
        r2 = copy(chunk(rout, opp), chunk(rin, my), r_s.at[2], r_r.at[2], opp)

        r0.wait()
        r1.wait()
        r2.wait()
        oh = (
            rpc_ref[...] == lax.broadcasted_iota(jnp.int32, (TOK, SLOTS), 1)
        ).astype(jnp.bfloat16)
        out_ref[...] = jnp.dot(oh, rin[...], preferred_element_type=jnp.float32)

    return pl.pallas_call(
        body,
        out_shape=jax.ShapeDtypeStruct((TOK, D), jnp.float32),
        in_specs=[pl.BlockSpec(memory_space=pltpu.VMEM)] * 6,
        out_specs=pl.BlockSpec(memory_space=pltpu.VMEM),
        scratch_shapes=[
            pltpu.VMEM((SLOTS, D), jnp.bfloat16),
            pltpu.VMEM((SLOTS, 1), jnp.bfloat16),
            pltpu.VMEM((SLOTS, D), jnp.bfloat16),
            pltpu.VMEM((SLOTS, 1), jnp.bfloat16),
            pltpu.VMEM((SLOTS, D), jnp.bfloat16),
            pltpu.VMEM((SLOTS, D), jnp.bfloat16),
            pltpu.VMEM((E_LOC, D, F), jnp.bfloat16),
            pltpu.VMEM((E_LOC, F, D), jnp.bfloat16),
            pltpu.SemaphoreType.DMA((3,)),
            pltpu.SemaphoreType.DMA((3,)),
            pltpu.SemaphoreType.DMA((3,)),
            pltpu.SemaphoreType.DMA((3,)),
            pltpu.SemaphoreType.DMA((3,)),
            pltpu.SemaphoreType.DMA((3,)),
        ],
        compiler_params=pltpu.CompilerParams(collective_id=0),
    )(x, assign2d, rp_col, rp_row, W1, W2)


# device time: 25825 ns/iter; 8.0702x vs baseline; 1.0743x over previous
import jax
import jax.numpy as jnp
from jax import lax
from jax.experimental import pallas as pl
from jax.experimental.pallas import tpu as pltpu

N_DEV = 4
TOK = 512
D = 512
F = 1024
E_LOC = 2
CAP = 160
SLOTS = N_DEV * CAP


def kernel(x, assign, W1, W2):
    assign2d = assign[:, None]

    def body(x_ref, a_ref, w1_ref, w2_ref, out_ref,
             xs, ts, xin, tin, rout, rin, w1b, w2b,
             d_s, d_r, t_s, t_r, r_s, r_r):
        my = lax.axis_index("i")
        left = (my + N_DEV - 1) % N_DEV
        right = (my + 1) % N_DEV
        opp = (my + 2) % N_DEV

        barrier_sem = pltpu.get_barrier_semaphore()
        for nbr in (left, right, opp):
            pl.semaphore_signal(
                barrier_sem, inc=1,
                device_id=(nbr,), device_id_type=pl.DeviceIdType.MESH,
            )
        pl.semaphore_wait(barrier_sem, 3)

        a_col = a_ref[...]
        dmask = (
            a_col // 2 == lax.broadcasted_iota(jnp.int32, (TOK, N_DEV), 1)
        ).astype(jnp.bfloat16)
        tri = (
            lax.broadcasted_iota(jnp.int32, (TOK, TOK), 0)
            >= lax.broadcasted_iota(jnp.int32, (TOK, TOK), 1)
        ).astype(jnp.bfloat16)
        pos = jnp.dot(tri, dmask, preferred_element_type=jnp.float32)
        dcap = (
            lax.broadcasted_iota(jnp.int32, (TOK, N_DEV), 1) * CAP
        ).astype(jnp.float32)
        rp = jnp.sum(
            dmask.astype(jnp.float32) * (dcap + pos - 1.0),
            axis=1, keepdims=True,
        ).astype(jnp.int32)

        oh = (
            rp == lax.broadcasted_iota(jnp.int32, (TOK, SLOTS), 1)
        ).astype(jnp.bfloat16)
        xb16 = x_ref[...].astype(jnp.bfloat16)
        xs[...] = lax.dot_general(
            oh, xb16, (((0,), (0,)), ((), ())),
            preferred_element_type=jnp.float32,
        ).astype(jnp.bfloat16)
        tagv = (a_col + 1).astype(jnp.bfloat16)
        ts[...] = lax.dot_general(
            oh, tagv, (((0,), (0,)), ((), ())),
            preferred_element_type=jnp.float32,
        ).astype(jnp.bfloat16)

        def copy(src_ref, dst_ref, send_sem, recv_sem, dst_dev):
            rdma = pltpu.make_async_remote_copy(
                src_ref=src_ref, dst_ref=dst_ref,
                send_sem=send_sem, recv_sem=recv_sem,
                device_id=(dst_dev,), device_id_type=pl.DeviceIdType.MESH,
            )
            rdma.start()
            return rdma

        def chunk(ref, s):
            return ref.at[pl.ds(s * CAP, CAP), :]

        d2 = copy(chunk(xs, opp), chunk(xin, my), d_s.at[2], d_r.at[2], opp)
        t2 = copy(chunk(ts, opp), chunk(tin, my), t_s.at[2], t_r.at[2], opp)
        d0 = copy(chunk(xs, right), chunk(xin, my), d_s.at[0], d_r.at[0], right)
        d1 = copy(chunk(xs, left), chunk(xin, my), d_s.at[1], d_r.at[1], left)
        t0 = copy(chunk(ts, right), chunk(tin, my), t_s.at[0], t_r.at[0], right)
        t1 = copy(chunk(ts, left), chunk(tin, my), t_s.at[1], t_r.at[1], left)

        w1b[...] = w1_ref[...].astype(jnp.bfloat16)
        w2b[...] = w2_ref[...].astype(jnp.bfloat16)

        def compute_chunk(s):
            xb = xin[pl.ds(s * CAP, CAP), :]
            tb = tin[pl.ds(s * CAP, CAP), :]
            acc = jnp.zeros((CAP, D), dtype=jnp.float32)
            for e in range(E_LOC):
                h_act = jnp.maximum(
                    jnp.dot(xb, w1b[e], preferred_element_type=jnp.float32),
                    0.0,
                ).astype(jnp.bfloat16)
                y = jnp.dot(h_act, w2b[e], preferred_element_type=jnp.float32)
                tag_e = (my * E_LOC + e + 1).astype(jnp.bfloat16)
                acc = acc + jnp.where(tb == tag_e, y, 0.0)
            rout[pl.ds(s * CAP, CAP), :] = acc.astype(jnp.bfloat16)

        xin[pl.ds(my * CAP, CAP), :] = xs[pl.ds(my * CAP, CAP), :]
        tin[pl.ds(my * CAP, CAP), :] = ts[pl.ds(my * CAP, CAP), :]
        compute_chunk(my)
        rin[pl.ds(my * CAP, CAP), :] = rout[pl.ds(my * CAP, CAP), :]

        d0.wait()
        t0.wait()
        compute_chunk(left)
        r0 = copy(chunk(rout, left), chunk(rin, my), r_s.at[0], r_r.at[0], left)

        d1.wait()
        t1.wait()
        compute_chunk(right)
        r1 = copy(chunk(rout, right), chunk(rin, my), r_s.at[1], r_r.at[1], right)

        d2.wait()
        t2.wait()
        compute_chunk(opp)
        r2 = copy(chunk(rout, opp), chunk(rin, my), r_s.at[2], r_r.at[2], opp)

        r0.wait()
        r1.wait()
        r2.wait()
        out_ref[...] = jnp.dot(oh, rin[...], preferred_element_type=jnp.float32)

    return pl.pallas_call(
        body,
        out_shape=jax.ShapeDtypeStruct((TOK, D), jnp.float32),
        in_specs=[pl.BlockSpec(memory_space=pltpu.VMEM)] * 4,
        out_specs=pl.BlockSpec(memory_space=pltpu.VMEM),
        scratch_shapes=[
            pltpu.VMEM((SLOTS, D), jnp.bfloat16),
            pltpu.VMEM((SLOTS, 1), jnp.bfloat16),
            pltpu.VMEM((SLOTS, D), jnp.bfloat16),
            pltpu.VMEM((SLOTS, 1), jnp.bfloat16),
            pltpu.VMEM((SLOTS, D), jnp.bfloat16),
            pltpu.VMEM((SLOTS, D), jnp.bfloat16),
            pltpu.VMEM((E_LOC, D, F), jnp.bfloat16),
            pltpu.VMEM((E_LOC, F, D), jnp.bfloat16),
            pltpu.SemaphoreType.DMA((3,)),
            pltpu.SemaphoreType.DMA((3,)),
            pltpu.SemaphoreType.DMA((3,)),
            pltpu.SemaphoreType.DMA((3,)),
            pltpu.SemaphoreType.DMA((3,)),
            pltpu.SemaphoreType.DMA((3,)),
        ],
        compiler_params=pltpu.CompilerParams(collective_id=0),
    )(x, assign2d, W1, W2)


# device time: 24293 ns/iter; 8.5791x vs baseline; 1.0631x over previous
import jax
import jax.numpy as jnp
from jax import lax
from jax.experimental import pallas as pl
from jax.experimental.pallas import tpu as pltpu

N_DEV = 4
TOK = 512
D = 512
F = 1024
E_LOC = 2
N_EXP = 8
CAP_E = 80
CAP = E_LOC * CAP_E
SLOTS = N_EXP * CAP_E


def kernel(x, assign, W1, W2):
    assign2d = assign[:, None]

    def body(x_ref, a_ref, w1_ref, w2_ref, out_ref,
             xs, xin, rout, rin, w1b, w2b,
             d_s, d_r, r_s, r_r):
        my = lax.axis_index("i")
        left = (my + N_DEV - 1) % N_DEV
        right = (my + 1) % N_DEV
        opp = (my + 2) % N_DEV

        barrier_sem = pltpu.get_barrier_semaphore()
        for nbr in (left, right, opp):
            pl.semaphore_signal(
                barrier_sem, inc=1,
                device_id=(nbr,), device_id_type=pl.DeviceIdType.MESH,
            )
        pl.semaphore_wait(barrier_sem, 3)

        a_col = a_ref[...]
        emask = (
            a_col == lax.broadcasted_iota(jnp.int32, (TOK, N_EXP), 1)
        ).astype(jnp.bfloat16)
        tri = (
            lax.broadcasted_iota(jnp.int32, (TOK, TOK), 0)
            >= lax.broadcasted_iota(jnp.int32, (TOK, TOK), 1)
        ).astype(jnp.bfloat16)
        pos = jnp.dot(tri, emask, preferred_element_type=jnp.float32)
        ecap = (
            lax.broadcasted_iota(jnp.int32, (TOK, N_EXP), 1) * CAP_E
        ).astype(jnp.float32)
        rp = jnp.sum(
            emask.astype(jnp.float32) * (ecap + pos - 1.0),
            axis=1, keepdims=True,
        ).astype(jnp.int32)

        oh = (
            rp == lax.broadcasted_iota(jnp.int32, (TOK, SLOTS), 1)
        ).astype(jnp.bfloat16)
        xb16 = x_ref[...].astype(jnp.bfloat16)
        xs[...] = lax.dot_general(
            oh, xb16, (((0,), (0,)), ((), ())),
            preferred_element_type=jnp.float32,
        ).astype(jnp.bfloat16)

        def copy(src_ref, dst_ref, send_sem, recv_sem, dst_dev):
            rdma = pltpu.make_async_remote_copy(
                src_ref=src_ref, dst_ref=dst_ref,
                send_sem=send_sem, recv_sem=recv_sem,
                device_id=(dst_dev,), device_id_type=pl.DeviceIdType.MESH,
            )
            rdma.start()
            return rdma

        def chunk(ref, s):
            return ref.at[pl.ds(s * CAP, CAP), :]

        d2 = copy(chunk(xs, opp), chunk(xin, my), d_s.at[2], d_r.at[2], opp)
        d0 = copy(chunk(xs, right), chunk(xin, my), d_s.at[0], d_r.at[0], right)
        d1 = copy(chunk(xs, left), chunk(xin, my), d_s.at[1], d_r.at[1], left)

        w1b[...] = w1_ref[...].astype(jnp.bfloat16)
        w2b[...] = w2_ref[...].astype(jnp.bfloat16)

        def compute_chunk(src_ref, s):
            base = s * CAP
            for e in range(E_LOC):
                xb = src_ref[pl.ds(base + e * CAP_E, CAP_E), :]
                h_act = jnp.maximum(
                    jnp.dot(xb, w1b[e], preferred_element_type=jnp.float32),
                    0.0,
                ).astype(jnp.bfloat16)
                y = jnp.dot(h_act, w2b[e], preferred_element_type=jnp.float32)
                rout[pl.ds(base + e * CAP_E, CAP_E), :] = y.astype(jnp.bfloat16)

        compute_chunk(xs, my)
        rin[pl.ds(my * CAP, CAP), :] = rout[pl.ds(my * CAP, CAP), :]

        d0.wait()
        compute_chunk(xin, left)
        r0 = copy(chunk(rout, left), chunk(rin, my), r_s.at[0], r_r.at[0], left)

        d1.wait()
        compute_chunk(xin, right)
        r1 = copy(chunk(rout, right), chunk(rin, my), r_s.at[1], r_r.at[1], right)

        d2.wait()
        compute_chunk(xin, opp)
        r2 = copy(chunk(rout, opp), chunk(rin, my), r_s.at[2], r_r.at[2], opp)

        r0.wait()
        r1.wait()
        r2.wait()
        out_ref[...] = jnp.dot(oh, rin[...], preferred_element_type=jnp.float32)

    return pl.pallas_call(
        body,
        out_shape=jax.ShapeDtypeStruct((TOK, D), jnp.float32),
        in_specs=[pl.BlockSpec(memory_space=pltpu.VMEM)] * 4,
        out_specs=pl.BlockSpec(memory_space=pltpu.VMEM),
        scratch_shapes=[
            pltpu.VMEM((SLOTS, D), jnp.bfloat16),
            pltpu.VMEM((SLOTS, D), jnp.bfloat16),
            pltpu.VMEM((SLOTS, D), jnp.bfloat16),
            pltpu.VMEM((SLOTS, D), jnp.bfloat16),
            pltpu.VMEM((E_LOC, D, F), jnp.bfloat16),
            pltpu.VMEM((E_LOC, F, D), jnp.bfloat16),
            pltpu.SemaphoreType.DMA((3,)),
            pltpu.SemaphoreType.DMA((3,)),
            pltpu.SemaphoreType.DMA((3,)),
            pltpu.SemaphoreType.DMA((3,)),
        ],
        compiler_params=pltpu.CompilerParams(collective_id=0),
    )(x, assign2d, W1, W2)
